# baseline (speedup 1.0000x reference)
import functools

import jax
import jax.numpy as jnp
from jax import lax
from jax.experimental import pallas as pl
from jax.experimental.pallas import tpu as pltpu
from jax.experimental.pallas import tpu_sc as plsc

N_NODES = 100000
N_EDGES = 1600000
N_PLANS = 1024

NC, NS, LANES = 2, 16, 16
NW = NC * NS
C = 128
K = 4
G = 100
EPT = G * K * C
EP = NW * EPT
PAD_NODE = N_NODES

NPAD = 100096
RPT = NPAD // NS

BCH = 25
BPTP = BCH * C
PAD_PLAN = N_PLANS
CPAD = 1152
CRPT = CPAD // NS

BN = 2000
GRID_N = N_NODES // BN

_MESH = plsc.VectorSubcoreMesh(
    core_axis_name="c", subcore_axis_name="s", num_cores=NC, num_subcores=NS)
_MESH1 = plsc.VectorSubcoreMesh(
    core_axis_name="c", subcore_axis_name="s", num_cores=1, num_subcores=NS)
G1 = 2 * G


@functools.partial(
    pl.kernel,
    out_type=(jax.ShapeDtypeStruct((NC, NPAD, LANES), jnp.float32),
              jax.ShapeDtypeStruct((NC, CPAD, LANES), jnp.float32)),
    mesh=_MESH,
    scratch_types=[
        pltpu.VMEM_SHARED((NPAD, LANES), jnp.float32),
        pltpu.VMEM_SHARED((CPAD, LANES), jnp.float32),
        pltpu.VMEM((2, K, C), jnp.int32),
        pltpu.VMEM((BCH, C), jnp.int32),
        pltpu.VMEM((C, LANES), jnp.float32),
        pltpu.SemaphoreType.DMA,
        pltpu.SemaphoreType.DMA,
    ],
    compiler_params=pltpu.CompilerParams(use_tc_tiling_on_sc=False),
)
def _deg_kernel(dst_hbm, bidx_hbm, zeros_hbm, ones_hbm,
                degp_hbm, cntp_hbm,
                dacc, cacc, dst_v, bidx_v, ones_v, ssemA, ssemB):
    c = lax.axis_index("c")
    s = lax.axis_index("s")
    wid = c * NS + s
    pltpu.sync_copy(ones_hbm, ones_v)
    pltpu.sync_copy(zeros_hbm, dacc.at[pl.ds(s * RPT, RPT)])
    pltpu.sync_copy(zeros_hbm.at[pl.ds(0, CRPT)], cacc.at[pl.ds(s * CRPT, CRPT)])
    plsc.subcore_barrier()

    def stage(g, slot):
        pltpu.sync_copy(dst_hbm.at[wid, g], dst_v.at[slot])

    def fire(slot, sem):
        return [pltpu.async_copy(ones_v, dacc.at[dst_v.at[slot, j]], sem,
                                 add=True)
                for j in range(K)]

    def drain(slot, sem):
        for j in range(K):
            pltpu.make_async_copy(ones_v, dacc.at[dst_v.at[slot, j]],
                                  sem).wait()

    stage(0, 0)
    fire(0, ssemA)
    stage(1, 1)
    fire(1, ssemB)

    def g_body(i, carry):
        drain(0, ssemA)
        stage(2 * i, 0)
        fire(0, ssemA)
        drain(1, ssemB)
        stage(2 * i + 1, 1)
        fire(1, ssemB)
        return carry

    lax.fori_loop(1, G // 2, g_body, 0)
    drain(0, ssemA)
    drain(1, ssemB)

    pltpu.sync_copy(bidx_hbm.at[wid], bidx_v)
    cdescs = [pltpu.async_copy(ones_v, cacc.at[bidx_v.at[j]], ssemA, add=True)
              for j in range(BCH)]
    for d in cdescs:
        d.wait()

    plsc.subcore_barrier()
    pltpu.sync_copy(dacc.at[pl.ds(s * RPT, RPT)],
                    degp_hbm.at[c, pl.ds(s * RPT, RPT)])
    pltpu.sync_copy(cacc.at[pl.ds(s * CRPT, CRPT)],
                    cntp_hbm.at[c, pl.ds(s * CRPT, CRPT)])


@functools.partial(
    pl.kernel,
    out_type=jax.ShapeDtypeStruct((NPAD, LANES), jnp.float32),
    mesh=_MESH1,
    scratch_types=[
        pltpu.VMEM_SHARED((NPAD, LANES), jnp.float32),
        pltpu.VMEM((2, K, C), jnp.int32),
        pltpu.VMEM((2, K, C), jnp.int32),
        pltpu.VMEM((2, K, C, LANES), jnp.float32),
        pltpu.SemaphoreType.DMA,
        pltpu.SemaphoreType.DMA,
        pltpu.SemaphoreType.DMA,
        pltpu.SemaphoreType.DMA,
    ],
    compiler_params=pltpu.CompilerParams(use_tc_tiling_on_sc=False),
)
def _agg_kernel(tbl_hbm, src_hbm, dst_hbm, zeros_hbm,
                out_hbm,
                acc, src_v, dst_v, rows_v, gsemA, gsemB, ssemA, ssemB):
    s = lax.axis_index("s")

    def stage(g, slot):
        pltpu.sync_copy(src_hbm.at[s, g], src_v.at[slot])
        pltpu.sync_copy(dst_hbm.at[s, g], dst_v.at[slot])

    def fire_gathers(slot, gsem):
        return [pltpu.async_copy(tbl_hbm.at[src_v.at[slot, j]],
                                 rows_v.at[slot, j], gsem)
                for j in range(K)]

    def fire_scatters(slot, ssem):
        return [pltpu.async_copy(rows_v.at[slot, j],
                                 acc.at[dst_v.at[slot, j]], ssem, add=True)
                for j in range(K)]

    pltpu.sync_copy(zeros_hbm, acc.at[pl.ds(s * RPT, RPT)])
    plsc.subcore_barrier()
    stage(0, 0)
    fire_gathers(0, gsemA)

    def body(i, carry):
        stage(2 * i + 1, 1)
        gdB = fire_gathers(1, gsemB)
        gdA_wait = [pltpu.make_async_copy(tbl_hbm.at[src_v.at[0, j]],
                                          rows_v.at[0, j], gsemA)
                    for j in range(K)]
        for d in gdA_wait:
            d.wait()
        sdA = fire_scatters(0, ssemA)
        for d in sdA:
            d.wait()

        @pl.when(i < G1 // 2 - 1)
        def _():
            stage(2 * i + 2, 0)
            fire_gathers(0, gsemA)

        for d in gdB:
            d.wait()
        sdB = fire_scatters(1, ssemB)
        for d in sdB:
            d.wait()
        return carry

    lax.fori_loop(0, G1 // 2, body, 0)
    plsc.subcore_barrier()
    pltpu.sync_copy(acc.at[pl.ds(s * RPT, RPT)],
                    out_hbm.at[pl.ds(s * RPT, RPT)])


def _enc_body(x_ref, degp_ref, w1_ref, b1_ref, w2_ref, b2_ref, w3_ref, b3_ref,
              ha_ref, hb_ref, dinv_ref):
    degp = degp_ref[...]
    deg = degp[0, :, 0] + degp[1, :, 0]
    dinv = lax.rsqrt(jnp.maximum(deg, 1.0))
    h = jnp.maximum(jnp.dot(x_ref[...], w1_ref[...],
                            preferred_element_type=jnp.float32) + b1_ref[...], 0.0)
    h = jnp.maximum(jnp.dot(h, w2_ref[...],
                            preferred_element_type=jnp.float32) + b2_ref[...], 0.0)
    h = jnp.dot(h, w3_ref[...], preferred_element_type=jnp.float32) + b3_ref[...]
    hp = h * dinv[:, None]
    ha_ref[...] = hp[:, :LANES]
    hb_ref[...] = hp[:, LANES:]
    dinv_ref[...] = jnp.broadcast_to(dinv[:, None], (BN, LANES))


def _l1_body(pa_ref, pb_ref, dinv_ref, w_ref, b_ref, o0, o1, o2, o3):
    agg = jnp.concatenate([pa_ref[...], pb_ref[...]], axis=1)
    dinv = dinv_ref[...][:, 0]
    h = jnp.maximum(jnp.dot(agg * dinv[:, None], w_ref[...],
                            preferred_element_type=jnp.float32) + b_ref[...], 0.0)
    hp = h * dinv[:, None]
    o0[...] = hp[:, 0:16]
    o1[...] = hp[:, 16:32]
    o2[...] = hp[:, 32:48]
    o3[...] = hp[:, 48:64]


def _l2_body(p0_ref, p1_ref, p2_ref, p3_ref, dinv_ref, bidx_ref, w_ref, b_ref,
             cntp_ref, hw1_ref, hb1_ref, hw2_ref, hb2_ref, hw3_ref, hb3_ref,
             pool_ref, cost_ref):
    i = pl.program_id(0)
    agg = jnp.concatenate([p0_ref[...], p1_ref[...], p2_ref[...], p3_ref[...]],
                          axis=1)
    dinv = dinv_ref[...][:, 0]
    h2 = jnp.maximum(jnp.dot(agg * dinv[:, None], w_ref[...],
                             preferred_element_type=jnp.float32) + b_ref[...], 0.0)
    plans = lax.broadcasted_iota(jnp.int32, (N_PLANS, BN), 0)
    onehot = (plans == bidx_ref[0, 0][None, :]).astype(jnp.float32)
    contrib = jnp.dot(onehot, h2, preferred_element_type=jnp.float32)

    @pl.when(i == 0)
    def _():
        pool_ref[...] = contrib

    @pl.when(i > 0)
    def _():
        pool_ref[...] += contrib

    @pl.when(i == GRID_N - 1)
    def _():
        cntp = cntp_ref[...]
        counts = cntp[0, :, 0] + cntp[1, :, 0]
        emb = pool_ref[...] / jnp.maximum(counts, 1.0)[:, None]
        z = jnp.maximum(jnp.dot(emb, hw1_ref[...],
                                preferred_element_type=jnp.float32) + hb1_ref[...], 0.0)
        z = jnp.maximum(jnp.dot(z, hw2_ref[...],
                                preferred_element_type=jnp.float32) + hb2_ref[...], 0.0)
        cst = jnp.dot(z, hw3_ref[...],
                      preferred_element_type=jnp.float32) + hb3_ref[...]
        cost_ref[...] = jnp.maximum(cst, 0.0) + jnp.log1p(jnp.exp(-jnp.abs(cst)))


def _full(shape):
    return pl.BlockSpec(shape, lambda i: tuple(0 for _ in shape))


def kernel(node_features, edge_index, edge_types, batch_idx,
           ne_W1, ne_b1, ne_W2, ne_b2, ne_W3, ne_b3,
           g_W1, g_b1, g_W2, g_b2,
           h_W1, h_b1, h_W2, h_b2, h_W3, h_b3):
    del edge_types
    src = edge_index[0].astype(jnp.int32)
    dst = edge_index[1].astype(jnp.int32)
    epad = PAD_NODE + (jnp.arange(EP - N_EDGES, dtype=jnp.int32) % (NPAD - N_NODES))
    src_flat = jnp.concatenate([src, epad])
    dst_flat = jnp.concatenate([dst, epad])
    src_r = src_flat.reshape(NW, G, K, C)
    dst_r = dst_flat.reshape(NW, G, K, C)
    src_r1 = src_flat.reshape(NS, G1, K, C)
    dst_r1 = dst_flat.reshape(NS, G1, K, C)
    bpad = PAD_PLAN + (jnp.arange(NW * BPTP - N_NODES, dtype=jnp.int32)
                       % (CPAD - N_PLANS))
    bidx_r = jnp.concatenate([batch_idx.astype(jnp.int32), bpad]).reshape(NW, BCH, C)
    batch2d = batch_idx.astype(jnp.int32).reshape(GRID_N, 1, BN)
    zeros_blk = jnp.zeros((RPT, LANES), jnp.float32)
    ones_blk = jnp.ones((C, LANES), jnp.float32)

    degp, cntp = _deg_kernel(dst_r, bidx_r, zeros_blk, ones_blk)

    h0a, h0b, dinv2d = pl.pallas_call(
        _enc_body,
        grid=(GRID_N,),
        in_specs=[
            pl.BlockSpec((BN, 50), lambda i: (i, 0)),
            pl.BlockSpec((NC, BN, LANES), lambda i: (0, i, 0)),
            _full((50, 128)), _full((128,)),
            _full((128, 64)), _full((64,)),
            _full((64, 32)), _full((32,)),
        ],
        out_specs=[
            pl.BlockSpec((BN, LANES), lambda i: (i, 0)),
            pl.BlockSpec((BN, LANES), lambda i: (i, 0)),
            pl.BlockSpec((BN, LANES), lambda i: (i, 0)),
        ],
        out_shape=[
            jax.ShapeDtypeStruct((NPAD, LANES), jnp.float32),
            jax.ShapeDtypeStruct((NPAD, LANES), jnp.float32),
            jax.ShapeDtypeStruct((NPAD, LANES), jnp.float32),
        ],
    )(node_features, degp, ne_W1, ne_b1, ne_W2, ne_b2, ne_W3, ne_b3)

    p1a = _agg_kernel(h0a, src_r1, dst_r1, zeros_blk)
    p1b = _agg_kernel(h0b, src_r1, dst_r1, zeros_blk)

    h1c = pl.pallas_call(
        _l1_body,
        grid=(GRID_N,),
        in_specs=[
            pl.BlockSpec((BN, LANES), lambda i: (i, 0)),
            pl.BlockSpec((BN, LANES), lambda i: (i, 0)),
            pl.BlockSpec((BN, LANES), lambda i: (i, 0)),
            _full((32, 64)), _full((64,)),
        ],
        out_specs=[pl.BlockSpec((BN, LANES), lambda i: (i, 0))] * 4,
        out_shape=[jax.ShapeDtypeStruct((NPAD, LANES), jnp.float32)] * 4,
    )(p1a, p1b, dinv2d, g_W1, g_b1)

    p2 = [_agg_kernel(t, src_r1, dst_r1, zeros_blk) for t in h1c]

    _, cost = pl.pallas_call(
        _l2_body,
        grid=(GRID_N,),
        in_specs=[
            pl.BlockSpec((BN, LANES), lambda i: (i, 0)),
            pl.BlockSpec((BN, LANES), lambda i: (i, 0)),
            pl.BlockSpec((BN, LANES), lambda i: (i, 0)),
            pl.BlockSpec((BN, LANES), lambda i: (i, 0)),
            pl.BlockSpec((BN, LANES), lambda i: (i, 0)),
            pl.BlockSpec((1, 1, BN), lambda i: (i, 0, 0)),
            _full((64, 64)), _full((64,)),
            pl.BlockSpec((NC, N_PLANS, LANES), lambda i: (0, 0, 0)),
            _full((64, 32)), _full((32,)),
            _full((32, 16)), _full((16,)),
            _full((16, 1)), _full((1,)),
        ],
        out_specs=[
            pl.BlockSpec((N_PLANS, 64), lambda i: (0, 0)),
            pl.BlockSpec((N_PLANS, 1), lambda i: (0, 0)),
        ],
        out_shape=[
            jax.ShapeDtypeStruct((N_PLANS, 64), jnp.float32),
            jax.ShapeDtypeStruct((N_PLANS, 1), jnp.float32),
        ],
    )(p2[0], p2[1], p2[2], p2[3], dinv2d, batch2d, g_W2, g_b2, cntp,
      h_W1, h_b1, h_W2, h_b2, h_W3, h_b3)
    return cost


# speedup vs baseline: 1.3383x; 1.3383x over previous
import functools

import jax
import jax.numpy as jnp
from jax import lax
from jax.experimental import pallas as pl
from jax.experimental.pallas import tpu as pltpu
from jax.experimental.pallas import tpu_sc as plsc

N_NODES = 100000
N_EDGES = 1600000
N_PLANS = 1024

NC, NS, LANES = 2, 16, 16
NW = NC * NS
C = 128
K = 4
G = 100
EPT = G * K * C
EP = NW * EPT
PAD_NODE = N_NODES

NPAD = 100096
RPT = NPAD // NS

BCH = 25
BPTP = BCH * C
PAD_PLAN = N_PLANS
CPAD = 1152
CRPT = CPAD // NS

BN = 2000
GRID_N = N_NODES // BN

_MESH = plsc.VectorSubcoreMesh(
    core_axis_name="c", subcore_axis_name="s", num_cores=NC, num_subcores=NS)


@functools.partial(
    pl.kernel,
    out_type=(jax.ShapeDtypeStruct((NC, NPAD, LANES), jnp.float32),
              jax.ShapeDtypeStruct((NC, CPAD, LANES), jnp.float32)),
    mesh=_MESH,
    scratch_types=[
        pltpu.VMEM_SHARED((NPAD, LANES), jnp.float32),
        pltpu.VMEM_SHARED((CPAD, LANES), jnp.float32),
        pltpu.VMEM((2, K, C), jnp.int32),
        pltpu.VMEM((BCH, C), jnp.int32),
        pltpu.VMEM((C, LANES), jnp.float32),
        pltpu.SemaphoreType.DMA,
        pltpu.SemaphoreType.DMA,
    ],
    compiler_params=pltpu.CompilerParams(use_tc_tiling_on_sc=False),
)
def _deg_kernel(dst_hbm, bidx_hbm, zeros_hbm, ones_hbm,
                degp_hbm, cntp_hbm,
                dacc, cacc, dst_v, bidx_v, ones_v, ssemA, ssemB):
    c = lax.axis_index("c")
    s = lax.axis_index("s")
    wid = c * NS + s
    pltpu.sync_copy(ones_hbm, ones_v)
    pltpu.sync_copy(zeros_hbm, dacc.at[pl.ds(s * RPT, RPT)])
    pltpu.sync_copy(zeros_hbm.at[pl.ds(0, CRPT)], cacc.at[pl.ds(s * CRPT, CRPT)])
    plsc.subcore_barrier()

    def stage(g, slot):
        pltpu.sync_copy(dst_hbm.at[wid, g], dst_v.at[slot])

    def fire(slot, sem):
        return [pltpu.async_copy(ones_v, dacc.at[dst_v.at[slot, j]], sem,
                                 add=True)
                for j in range(K)]

    def drain(slot, sem):
        for j in range(K):
            pltpu.make_async_copy(ones_v, dacc.at[dst_v.at[slot, j]],
                                  sem).wait()

    stage(0, 0)
    fire(0, ssemA)
    stage(1, 1)
    fire(1, ssemB)

    def g_body(i, carry):
        drain(0, ssemA)
        stage(2 * i, 0)
        fire(0, ssemA)
        drain(1, ssemB)
        stage(2 * i + 1, 1)
        fire(1, ssemB)
        return carry

    lax.fori_loop(1, G // 2, g_body, 0)
    drain(0, ssemA)
    drain(1, ssemB)

    pltpu.sync_copy(bidx_hbm.at[wid], bidx_v)
    cdescs = [pltpu.async_copy(ones_v, cacc.at[bidx_v.at[j]], ssemA, add=True)
              for j in range(BCH)]
    for d in cdescs:
        d.wait()

    plsc.subcore_barrier()
    pltpu.sync_copy(dacc.at[pl.ds(s * RPT, RPT)],
                    degp_hbm.at[c, pl.ds(s * RPT, RPT)])
    pltpu.sync_copy(cacc.at[pl.ds(s * CRPT, CRPT)],
                    cntp_hbm.at[c, pl.ds(s * CRPT, CRPT)])


@functools.partial(
    pl.kernel,
    out_type=jax.ShapeDtypeStruct((NC, NPAD, LANES), jnp.float32),
    mesh=_MESH,
    scratch_types=[
        pltpu.VMEM_SHARED((NPAD, LANES), jnp.float32),
        pltpu.VMEM((2, K, C), jnp.int32),
        pltpu.VMEM((2, K, C), jnp.int32),
        pltpu.VMEM((2, K, C, LANES), jnp.float32),
        pltpu.SemaphoreType.DMA,
        pltpu.SemaphoreType.DMA,
        pltpu.SemaphoreType.DMA,
        pltpu.SemaphoreType.DMA,
    ],
    compiler_params=pltpu.CompilerParams(use_tc_tiling_on_sc=False),
)
def _agg_kernel(tbl_hbm, src_hbm, dst_hbm, zeros_hbm,
                out_hbm,
                acc, src_v, dst_v, rows_v, gsemA, gsemB, ssemA, ssemB):
    c = lax.axis_index("c")
    s = lax.axis_index("s")
    wid = c * NS + s

    def stage(g, slot):
        pltpu.sync_copy(src_hbm.at[wid, g], src_v.at[slot])
        pltpu.sync_copy(dst_hbm.at[wid, g], dst_v.at[slot])

    def fire_gathers(slot, gsem):
        return [pltpu.async_copy(tbl_hbm.at[src_v.at[slot, j]],
                                 rows_v.at[slot, j], gsem)
                for j in range(K)]

    def fire_scatters(slot, ssem):
        return [pltpu.async_copy(rows_v.at[slot, j],
                                 acc.at[dst_v.at[slot, j]], ssem, add=True)
                for j in range(K)]

    pltpu.sync_copy(zeros_hbm, acc.at[pl.ds(s * RPT, RPT)])
    plsc.subcore_barrier()
    stage(0, 0)
    fire_gathers(0, gsemA)

    def body(i, carry):
        stage(2 * i + 1, 1)
        gdB = fire_gathers(1, gsemB)
        gdA_wait = [pltpu.make_async_copy(tbl_hbm.at[src_v.at[0, j]],
                                          rows_v.at[0, j], gsemA)
                    for j in range(K)]
        for d in gdA_wait:
            d.wait()
        sdA = fire_scatters(0, ssemA)
        for d in sdA:
            d.wait()

        @pl.when(i < G // 2 - 1)
        def _():
            stage(2 * i + 2, 0)
            fire_gathers(0, gsemA)

        for d in gdB:
            d.wait()
        sdB = fire_scatters(1, ssemB)
        for d in sdB:
            d.wait()
        return carry

    lax.fori_loop(0, G // 2, body, 0)
    plsc.subcore_barrier()
    pltpu.sync_copy(acc.at[pl.ds(s * RPT, RPT)],
                    out_hbm.at[c, pl.ds(s * RPT, RPT)])


def _enc_body(x_ref, degp_ref, w1_ref, b1_ref, w2_ref, b2_ref, w3_ref, b3_ref,
              ha_ref, hb_ref, dinv_ref):
    degp = degp_ref[...]
    deg = degp[0, :, 0] + degp[1, :, 0]
    dinv = lax.rsqrt(jnp.maximum(deg, 1.0))
    h = jnp.maximum(jnp.dot(x_ref[...], w1_ref[...],
                            preferred_element_type=jnp.float32) + b1_ref[...], 0.0)
    h = jnp.maximum(jnp.dot(h, w2_ref[...],
                            preferred_element_type=jnp.float32) + b2_ref[...], 0.0)
    h = jnp.dot(h, w3_ref[...], preferred_element_type=jnp.float32) + b3_ref[...]
    hp = h * dinv[:, None]
    ha_ref[...] = hp[:, :LANES]
    hb_ref[...] = hp[:, LANES:]
    dinv_ref[...] = jnp.broadcast_to(dinv[:, None], (BN, LANES))


def _l1_body(pa_ref, pb_ref, dinv_ref, w_ref, b_ref, o0, o1, o2, o3):
    pa = pa_ref[...]
    pb = pb_ref[...]
    agg = jnp.concatenate([pa[0] + pa[1], pb[0] + pb[1]], axis=1)
    dinv = dinv_ref[...][:, 0]
    h = jnp.maximum(jnp.dot(agg * dinv[:, None], w_ref[...],
                            preferred_element_type=jnp.float32) + b_ref[...], 0.0)
    hp = h * dinv[:, None]
    o0[...] = hp[:, 0:16]
    o1[...] = hp[:, 16:32]
    o2[...] = hp[:, 32:48]
    o3[...] = hp[:, 48:64]


def _l2_body(p0_ref, p1_ref, p2_ref, p3_ref, dinv_ref, bidx_ref, w_ref, b_ref,
             cntp_ref, hw1_ref, hb1_ref, hw2_ref, hb2_ref, hw3_ref, hb3_ref,
             pool_ref, cost_ref):
    i = pl.program_id(0)
    parts = [p0_ref[...], p1_ref[...], p2_ref[...], p3_ref[...]]
    agg = jnp.concatenate([p[0] + p[1] for p in parts], axis=1)
    dinv = dinv_ref[...][:, 0]
    h2 = jnp.maximum(jnp.dot(agg * dinv[:, None], w_ref[...],
                             preferred_element_type=jnp.float32) + b_ref[...], 0.0)
    plans = lax.broadcasted_iota(jnp.int32, (N_PLANS, BN), 0)
    onehot = (plans == bidx_ref[0, 0][None, :]).astype(jnp.float32)
    contrib = jnp.dot(onehot, h2, preferred_element_type=jnp.float32)

    @pl.when(i == 0)
    def _():
        pool_ref[...] = contrib

    @pl.when(i > 0)
    def _():
        pool_ref[...] += contrib

    @pl.when(i == GRID_N - 1)
    def _():
        cntp = cntp_ref[...]
        counts = cntp[0, :, 0] + cntp[1, :, 0]
        emb = pool_ref[...] / jnp.maximum(counts, 1.0)[:, None]
        z = jnp.maximum(jnp.dot(emb, hw1_ref[...],
                                preferred_element_type=jnp.float32) + hb1_ref[...], 0.0)
        z = jnp.maximum(jnp.dot(z, hw2_ref[...],
                                preferred_element_type=jnp.float32) + hb2_ref[...], 0.0)
        cst = jnp.dot(z, hw3_ref[...],
                      preferred_element_type=jnp.float32) + hb3_ref[...]
        cost_ref[...] = jnp.maximum(cst, 0.0) + jnp.log1p(jnp.exp(-jnp.abs(cst)))


def _full(shape):
    return pl.BlockSpec(shape, lambda i: tuple(0 for _ in shape))


def kernel(node_features, edge_index, edge_types, batch_idx,
           ne_W1, ne_b1, ne_W2, ne_b2, ne_W3, ne_b3,
           g_W1, g_b1, g_W2, g_b2,
           h_W1, h_b1, h_W2, h_b2, h_W3, h_b3):
    del edge_types
    src = edge_index[0].astype(jnp.int32)
    dst = edge_index[1].astype(jnp.int32)
    epad = PAD_NODE + (jnp.arange(EP - N_EDGES, dtype=jnp.int32) % (NPAD - N_NODES))
    src_flat = jnp.concatenate([src, epad])
    dst_flat = jnp.concatenate([dst, epad])
    src_r = src_flat.reshape(NW, G, K, C)
    dst_r = dst_flat.reshape(NW, G, K, C)
    bpad = PAD_PLAN + (jnp.arange(NW * BPTP - N_NODES, dtype=jnp.int32)
                       % (CPAD - N_PLANS))
    bidx_r = jnp.concatenate([batch_idx.astype(jnp.int32), bpad]).reshape(NW, BCH, C)
    batch2d = batch_idx.astype(jnp.int32).reshape(GRID_N, 1, BN)
    zeros_blk = jnp.zeros((RPT, LANES), jnp.float32)
    ones_blk = jnp.ones((C, LANES), jnp.float32)

    degp, cntp = _deg_kernel(dst_r, bidx_r, zeros_blk, ones_blk)

    h0a, h0b, dinv2d = pl.pallas_call(
        _enc_body,
        grid=(GRID_N,),
        in_specs=[
            pl.BlockSpec((BN, 50), lambda i: (i, 0)),
            pl.BlockSpec((NC, BN, LANES), lambda i: (0, i, 0)),
            _full((50, 128)), _full((128,)),
            _full((128, 64)), _full((64,)),
            _full((64, 32)), _full((32,)),
        ],
        out_specs=[
            pl.BlockSpec((BN, LANES), lambda i: (i, 0)),
            pl.BlockSpec((BN, LANES), lambda i: (i, 0)),
            pl.BlockSpec((BN, LANES), lambda i: (i, 0)),
        ],
        out_shape=[
            jax.ShapeDtypeStruct((NPAD, LANES), jnp.float32),
            jax.ShapeDtypeStruct((NPAD, LANES), jnp.float32),
            jax.ShapeDtypeStruct((NPAD, LANES), jnp.float32),
        ],
    )(node_features, degp, ne_W1, ne_b1, ne_W2, ne_b2, ne_W3, ne_b3)

    p1a = _agg_kernel(h0a, src_r, dst_r, zeros_blk)
    p1b = _agg_kernel(h0b, src_r, dst_r, zeros_blk)

    h1c = pl.pallas_call(
        _l1_body,
        grid=(GRID_N,),
        in_specs=[
            pl.BlockSpec((NC, BN, LANES), lambda i: (0, i, 0)),
            pl.BlockSpec((NC, BN, LANES), lambda i: (0, i, 0)),
            pl.BlockSpec((BN, LANES), lambda i: (i, 0)),
            _full((32, 64)), _full((64,)),
        ],
        out_specs=[pl.BlockSpec((BN, LANES), lambda i: (i, 0))] * 4,
        out_shape=[jax.ShapeDtypeStruct((NPAD, LANES), jnp.float32)] * 4,
    )(p1a, p1b, dinv2d, g_W1, g_b1)

    p2 = [_agg_kernel(t, src_r, dst_r, zeros_blk) for t in h1c]

    _, cost = pl.pallas_call(
        _l2_body,
        grid=(GRID_N,),
        in_specs=[
            pl.BlockSpec((NC, BN, LANES), lambda i: (0, i, 0)),
            pl.BlockSpec((NC, BN, LANES), lambda i: (0, i, 0)),
            pl.BlockSpec((NC, BN, LANES), lambda i: (0, i, 0)),
            pl.BlockSpec((NC, BN, LANES), lambda i: (0, i, 0)),
            pl.BlockSpec((BN, LANES), lambda i: (i, 0)),
            pl.BlockSpec((1, 1, BN), lambda i: (i, 0, 0)),
            _full((64, 64)), _full((64,)),
            pl.BlockSpec((NC, N_PLANS, LANES), lambda i: (0, 0, 0)),
            _full((64, 32)), _full((32,)),
            _full((32, 16)), _full((16,)),
            _full((16, 1)), _full((1,)),
        ],
        out_specs=[
            pl.BlockSpec((N_PLANS, 64), lambda i: (0, 0)),
            pl.BlockSpec((N_PLANS, 1), lambda i: (0, 0)),
        ],
        out_shape=[
            jax.ShapeDtypeStruct((N_PLANS, 64), jnp.float32),
            jax.ShapeDtypeStruct((N_PLANS, 1), jnp.float32),
        ],
    )(p2[0], p2[1], p2[2], p2[3], dinv2d, batch2d, g_W2, g_b2, cntp,
      h_W1, h_b1, h_W2, h_b2, h_W3, h_b3)
    return cost


# speedup vs baseline: 1.4293x; 1.0680x over previous
import functools

import jax
import jax.numpy as jnp
from jax import lax
from jax.experimental import pallas as pl
from jax.experimental.pallas import tpu as pltpu
from jax.experimental.pallas import tpu_sc as plsc

N_NODES = 100000
N_EDGES = 1600000
N_PLANS = 1024

NC, NS, LANES = 2, 16, 16
NW = NC * NS
C = 128
K = 5
G = 80
EPT = G * K * C
EP = NW * EPT
PAD_NODE = N_NODES

NPAD = 100096
RPT = NPAD // NS

BCH = 25
BPTP = BCH * C
PAD_PLAN = N_PLANS
CPAD = 1152
CRPT = CPAD // NS

BN = 2000
GRID_N = N_NODES // BN

_MESH = plsc.VectorSubcoreMesh(
    core_axis_name="c", subcore_axis_name="s", num_cores=NC, num_subcores=NS)


@functools.partial(
    pl.kernel,
    out_type=(jax.ShapeDtypeStruct((NC, NPAD, LANES), jnp.float32),
              jax.ShapeDtypeStruct((NC, CPAD, LANES), jnp.float32)),
    mesh=_MESH,
    scratch_types=[
        pltpu.VMEM_SHARED((NPAD, LANES), jnp.float32),
        pltpu.VMEM_SHARED((CPAD, LANES), jnp.float32),
        pltpu.VMEM((2, K, C), jnp.int32),
        pltpu.VMEM((BCH, C), jnp.int32),
        pltpu.VMEM((C, LANES), jnp.float32),
        pltpu.SemaphoreType.DMA,
        pltpu.SemaphoreType.DMA,
    ],
    compiler_params=pltpu.CompilerParams(use_tc_tiling_on_sc=False),
)
def _deg_kernel(dst_hbm, bidx_hbm, zeros_hbm, ones_hbm,
                degp_hbm, cntp_hbm,
                dacc, cacc, dst_v, bidx_v, ones_v, ssemA, ssemB):
    c = lax.axis_index("c")
    s = lax.axis_index("s")
    wid = c * NS + s
    pltpu.sync_copy(ones_hbm, ones_v)
    pltpu.sync_copy(zeros_hbm, dacc.at[pl.ds(s * RPT, RPT)])
    pltpu.sync_copy(zeros_hbm.at[pl.ds(0, CRPT)], cacc.at[pl.ds(s * CRPT, CRPT)])
    plsc.subcore_barrier()

    def stage(g, slot):
        pltpu.sync_copy(dst_hbm.at[wid, g], dst_v.at[slot])

    def fire(slot, sem):
        return [pltpu.async_copy(ones_v, dacc.at[dst_v.at[slot, j]], sem,
                                 add=True)
                for j in range(K)]

    def drain(slot, sem):
        for j in range(K):
            pltpu.make_async_copy(ones_v, dacc.at[dst_v.at[slot, j]],
                                  sem).wait()

    stage(0, 0)
    fire(0, ssemA)
    stage(1, 1)
    fire(1, ssemB)

    def g_body(i, carry):
        drain(0, ssemA)
        stage(2 * i, 0)
        fire(0, ssemA)
        drain(1, ssemB)
        stage(2 * i + 1, 1)
        fire(1, ssemB)
        return carry

    lax.fori_loop(1, G // 2, g_body, 0)
    drain(0, ssemA)
    drain(1, ssemB)

    pltpu.sync_copy(bidx_hbm.at[wid], bidx_v)
    cdescs = [pltpu.async_copy(ones_v, cacc.at[bidx_v.at[j]], ssemA, add=True)
              for j in range(BCH)]
    for d in cdescs:
        d.wait()

    plsc.subcore_barrier()
    pltpu.sync_copy(dacc.at[pl.ds(s * RPT, RPT)],
                    degp_hbm.at[c, pl.ds(s * RPT, RPT)])
    pltpu.sync_copy(cacc.at[pl.ds(s * CRPT, CRPT)],
                    cntp_hbm.at[c, pl.ds(s * CRPT, CRPT)])


@functools.partial(
    pl.kernel,
    out_type=jax.ShapeDtypeStruct((NC, NPAD, LANES), jnp.float32),
    mesh=_MESH,
    scratch_types=[
        pltpu.VMEM_SHARED((NPAD, LANES), jnp.float32),
        pltpu.VMEM((2, K, C), jnp.int32),
        pltpu.VMEM((2, K, C), jnp.int32),
        pltpu.VMEM((2, K, C, LANES), jnp.float32),
        pltpu.SemaphoreType.DMA,
        pltpu.SemaphoreType.DMA,
        pltpu.SemaphoreType.DMA,
        pltpu.SemaphoreType.DMA,
    ],
    compiler_params=pltpu.CompilerParams(use_tc_tiling_on_sc=False),
)
def _agg_kernel(tbl_hbm, src_hbm, dst_hbm, zeros_hbm,
                out_hbm,
                acc, src_v, dst_v, rows_v, gsemA, gsemB, ssemA, ssemB):
    c = lax.axis_index("c")
    s = lax.axis_index("s")
    wid = c * NS + s

    def stage(g, slot):
        pltpu.sync_copy(src_hbm.at[wid, g], src_v.at[slot])
        pltpu.sync_copy(dst_hbm.at[wid, g], dst_v.at[slot])

    def fire_gathers(slot, gsem):
        return [pltpu.async_copy(tbl_hbm.at[src_v.at[slot, j]],
                                 rows_v.at[slot, j], gsem)
                for j in range(K)]

    def fire_scatters(slot, ssem):
        return [pltpu.async_copy(rows_v.at[slot, j],
                                 acc.at[dst_v.at[slot, j]], ssem, add=True)
                for j in range(K)]

    pltpu.sync_copy(zeros_hbm, acc.at[pl.ds(s * RPT, RPT)])
    plsc.subcore_barrier()
    stage(0, 0)
    fire_gathers(0, gsemA)

    def body(i, carry):
        stage(2 * i + 1, 1)
        gdB = fire_gathers(1, gsemB)
        gdA_wait = [pltpu.make_async_copy(tbl_hbm.at[src_v.at[0, j]],
                                          rows_v.at[0, j], gsemA)
                    for j in range(K)]
        for d in gdA_wait:
            d.wait()
        sdA = fire_scatters(0, ssemA)
        for d in sdA:
            d.wait()

        @pl.when(i < G // 2 - 1)
        def _():
            stage(2 * i + 2, 0)
            fire_gathers(0, gsemA)

        for d in gdB:
            d.wait()
        sdB = fire_scatters(1, ssemB)
        for d in sdB:
            d.wait()
        return carry

    lax.fori_loop(0, G // 2, body, 0)
    plsc.subcore_barrier()
    pltpu.sync_copy(acc.at[pl.ds(s * RPT, RPT)],
                    out_hbm.at[c, pl.ds(s * RPT, RPT)])


def _enc_body(x_ref, degp_ref, w1_ref, b1_ref, w2_ref, b2_ref, w3_ref, b3_ref,
              ha_ref, hb_ref, dinv_ref):
    degp = degp_ref[...]
    deg = degp[0, :, 0] + degp[1, :, 0]
    dinv = lax.rsqrt(jnp.maximum(deg, 1.0))
    h = jnp.maximum(jnp.dot(x_ref[...], w1_ref[...],
                            preferred_element_type=jnp.float32) + b1_ref[...], 0.0)
    h = jnp.maximum(jnp.dot(h, w2_ref[...],
                            preferred_element_type=jnp.float32) + b2_ref[...], 0.0)
    h = jnp.dot(h, w3_ref[...], preferred_element_type=jnp.float32) + b3_ref[...]
    hp = h * dinv[:, None]
    ha_ref[...] = hp[:, :LANES]
    hb_ref[...] = hp[:, LANES:]
    dinv_ref[...] = jnp.broadcast_to(dinv[:, None], (BN, LANES))


def _l1_body(pa_ref, pb_ref, dinv_ref, w_ref, b_ref, o0, o1, o2, o3):
    pa = pa_ref[...]
    pb = pb_ref[...]
    agg = jnp.concatenate([pa[0] + pa[1], pb[0] + pb[1]], axis=1)
    dinv = dinv_ref[...][:, 0]
    h = jnp.maximum(jnp.dot(agg * dinv[:, None], w_ref[...],
                            preferred_element_type=jnp.float32) + b_ref[...], 0.0)
    hp = h * dinv[:, None]
    o0[...] = hp[:, 0:16]
    o1[...] = hp[:, 16:32]
    o2[...] = hp[:, 32:48]
    o3[...] = hp[:, 48:64]


def _l2_body(p0_ref, p1_ref, p2_ref, p3_ref, dinv_ref, bidx_ref, w_ref, b_ref,
             cntp_ref, hw1_ref, hb1_ref, hw2_ref, hb2_ref, hw3_ref, hb3_ref,
             pool_ref, cost_ref):
    i = pl.program_id(0)
    parts = [p0_ref[...], p1_ref[...], p2_ref[...], p3_ref[...]]
    agg = jnp.concatenate([p[0] + p[1] for p in parts], axis=1)
    dinv = dinv_ref[...][:, 0]
    h2 = jnp.maximum(jnp.dot(agg * dinv[:, None], w_ref[...],
                             preferred_element_type=jnp.float32) + b_ref[...], 0.0)
    plans = lax.broadcasted_iota(jnp.int32, (N_PLANS, BN), 0)
    onehot = (plans == bidx_ref[0, 0][None, :]).astype(jnp.float32)
    contrib = jnp.dot(onehot, h2, preferred_element_type=jnp.float32)

    @pl.when(i == 0)
    def _():
        pool_ref[...] = contrib

    @pl.when(i > 0)
    def _():
        pool_ref[...] += contrib

    @pl.when(i == GRID_N - 1)
    def _():
        cntp = cntp_ref[...]
        counts = cntp[0, :, 0] + cntp[1, :, 0]
        emb = pool_ref[...] / jnp.maximum(counts, 1.0)[:, None]
        z = jnp.maximum(jnp.dot(emb, hw1_ref[...],
                                preferred_element_type=jnp.float32) + hb1_ref[...], 0.0)
        z = jnp.maximum(jnp.dot(z, hw2_ref[...],
                                preferred_element_type=jnp.float32) + hb2_ref[...], 0.0)
        cst = jnp.dot(z, hw3_ref[...],
                      preferred_element_type=jnp.float32) + hb3_ref[...]
        cost_ref[...] = jnp.maximum(cst, 0.0) + jnp.log1p(jnp.exp(-jnp.abs(cst)))


def _full(shape):
    return pl.BlockSpec(shape, lambda i: tuple(0 for _ in shape))


def kernel(node_features, edge_index, edge_types, batch_idx,
           ne_W1, ne_b1, ne_W2, ne_b2, ne_W3, ne_b3,
           g_W1, g_b1, g_W2, g_b2,
           h_W1, h_b1, h_W2, h_b2, h_W3, h_b3):
    del edge_types
    src = edge_index[0].astype(jnp.int32)
    dst = edge_index[1].astype(jnp.int32)
    epad = PAD_NODE + (jnp.arange(EP - N_EDGES, dtype=jnp.int32) % (NPAD - N_NODES))
    src_flat = jnp.concatenate([src, epad])
    dst_flat = jnp.concatenate([dst, epad])
    src_r = src_flat.reshape(NW, G, K, C)
    dst_r = dst_flat.reshape(NW, G, K, C)
    bpad = PAD_PLAN + (jnp.arange(NW * BPTP - N_NODES, dtype=jnp.int32)
                       % (CPAD - N_PLANS))
    bidx_r = jnp.concatenate([batch_idx.astype(jnp.int32), bpad]).reshape(NW, BCH, C)
    batch2d = batch_idx.astype(jnp.int32).reshape(GRID_N, 1, BN)
    zeros_blk = jnp.zeros((RPT, LANES), jnp.float32)
    ones_blk = jnp.ones((C, LANES), jnp.float32)

    degp, cntp = _deg_kernel(dst_r, bidx_r, zeros_blk, ones_blk)

    h0a, h0b, dinv2d = pl.pallas_call(
        _enc_body,
        grid=(GRID_N,),
        in_specs=[
            pl.BlockSpec((BN, 50), lambda i: (i, 0)),
            pl.BlockSpec((NC, BN, LANES), lambda i: (0, i, 0)),
            _full((50, 128)), _full((128,)),
            _full((128, 64)), _full((64,)),
            _full((64, 32)), _full((32,)),
        ],
        out_specs=[
            pl.BlockSpec((BN, LANES), lambda i: (i, 0)),
            pl.BlockSpec((BN, LANES), lambda i: (i, 0)),
            pl.BlockSpec((BN, LANES), lambda i: (i, 0)),
        ],
        out_shape=[
            jax.ShapeDtypeStruct((NPAD, LANES), jnp.float32),
            jax.ShapeDtypeStruct((NPAD, LANES), jnp.float32),
            jax.ShapeDtypeStruct((NPAD, LANES), jnp.float32),
        ],
    )(node_features, degp, ne_W1, ne_b1, ne_W2, ne_b2, ne_W3, ne_b3)

    p1a = _agg_kernel(h0a, src_r, dst_r, zeros_blk)
    p1b = _agg_kernel(h0b, src_r, dst_r, zeros_blk)

    h1c = pl.pallas_call(
        _l1_body,
        grid=(GRID_N,),
        in_specs=[
            pl.BlockSpec((NC, BN, LANES), lambda i: (0, i, 0)),
            pl.BlockSpec((NC, BN, LANES), lambda i: (0, i, 0)),
            pl.BlockSpec((BN, LANES), lambda i: (i, 0)),
            _full((32, 64)), _full((64,)),
        ],
        out_specs=[pl.BlockSpec((BN, LANES), lambda i: (i, 0))] * 4,
        out_shape=[jax.ShapeDtypeStruct((NPAD, LANES), jnp.float32)] * 4,
    )(p1a, p1b, dinv2d, g_W1, g_b1)

    p2 = [_agg_kernel(t, src_r, dst_r, zeros_blk) for t in h1c]

    _, cost = pl.pallas_call(
        _l2_body,
        grid=(GRID_N,),
        in_specs=[
            pl.BlockSpec((NC, BN, LANES), lambda i: (0, i, 0)),
            pl.BlockSpec((NC, BN, LANES), lambda i: (0, i, 0)),
            pl.BlockSpec((NC, BN, LANES), lambda i: (0, i, 0)),
            pl.BlockSpec((NC, BN, LANES), lambda i: (0, i, 0)),
            pl.BlockSpec((BN, LANES), lambda i: (i, 0)),
            pl.BlockSpec((1, 1, BN), lambda i: (i, 0, 0)),
            _full((64, 64)), _full((64,)),
            pl.BlockSpec((NC, N_PLANS, LANES), lambda i: (0, 0, 0)),
            _full((64, 32)), _full((32,)),
            _full((32, 16)), _full((16,)),
            _full((16, 1)), _full((1,)),
        ],
        out_specs=[
            pl.BlockSpec((N_PLANS, 64), lambda i: (0, 0)),
            pl.BlockSpec((N_PLANS, 1), lambda i: (0, 0)),
        ],
        out_shape=[
            jax.ShapeDtypeStruct((N_PLANS, 64), jnp.float32),
            jax.ShapeDtypeStruct((N_PLANS, 1), jnp.float32),
        ],
    )(p2[0], p2[1], p2[2], p2[3], dinv2d, batch2d, g_W2, g_b2, cntp,
      h_W1, h_b1, h_W2, h_b2, h_W3, h_b3)
    return cost


# speedup vs baseline: 1.5022x; 1.0510x over previous
import functools

import jax
import jax.numpy as jnp
from jax import lax
from jax.experimental import pallas as pl
from jax.experimental.pallas import tpu as pltpu
from jax.experimental.pallas import tpu_sc as plsc

N_NODES = 100000
N_EDGES = 1600000
N_PLANS = 1024

NC, NS, LANES = 2, 16, 16
NW = NC * NS
C = 128
K = 6
G = 66
EPT = G * K * C
EP = NW * EPT
PAD_NODE = N_NODES

NPAD = 100096
RPT = NPAD // NS

BCH = 25
BPTP = BCH * C
PAD_PLAN = N_PLANS
CPAD = 1152
CRPT = CPAD // NS

BN = 2000
GRID_N = N_NODES // BN

_MESH = plsc.VectorSubcoreMesh(
    core_axis_name="c", subcore_axis_name="s", num_cores=NC, num_subcores=NS)


@functools.partial(
    pl.kernel,
    out_type=(jax.ShapeDtypeStruct((NC, NPAD, LANES), jnp.float32),
              jax.ShapeDtypeStruct((NC, CPAD, LANES), jnp.float32)),
    mesh=_MESH,
    scratch_types=[
        pltpu.VMEM_SHARED((NPAD, LANES), jnp.float32),
        pltpu.VMEM_SHARED((CPAD, LANES), jnp.float32),
        pltpu.VMEM((2, K, C), jnp.int32),
        pltpu.VMEM((BCH, C), jnp.int32),
        pltpu.VMEM((C, LANES), jnp.float32),
        pltpu.SemaphoreType.DMA,
        pltpu.SemaphoreType.DMA,
    ],
    compiler_params=pltpu.CompilerParams(use_tc_tiling_on_sc=False),
)
def _deg_kernel(dst_hbm, bidx_hbm, zeros_hbm, ones_hbm,
                degp_hbm, cntp_hbm,
                dacc, cacc, dst_v, bidx_v, ones_v, ssemA, ssemB):
    c = lax.axis_index("c")
    s = lax.axis_index("s")
    wid = c * NS + s
    pltpu.sync_copy(ones_hbm, ones_v)
    pltpu.sync_copy(zeros_hbm, dacc.at[pl.ds(s * RPT, RPT)])
    pltpu.sync_copy(zeros_hbm.at[pl.ds(0, CRPT)], cacc.at[pl.ds(s * CRPT, CRPT)])
    plsc.subcore_barrier()

    def stage(g, slot):
        pltpu.sync_copy(dst_hbm.at[wid, g], dst_v.at[slot])

    def fire(slot, sem):
        return [pltpu.async_copy(ones_v, dacc.at[dst_v.at[slot, j]], sem,
                                 add=True)
                for j in range(K)]

    def drain(slot, sem):
        for j in range(K):
            pltpu.make_async_copy(ones_v, dacc.at[dst_v.at[slot, j]],
                                  sem).wait()

    stage(0, 0)
    fire(0, ssemA)
    stage(1, 1)
    fire(1, ssemB)

    def g_body(i, carry):
        drain(0, ssemA)
        stage(2 * i, 0)
        fire(0, ssemA)
        drain(1, ssemB)
        stage(2 * i + 1, 1)
        fire(1, ssemB)
        return carry

    lax.fori_loop(1, G // 2, g_body, 0)
    drain(0, ssemA)
    drain(1, ssemB)

    pltpu.sync_copy(bidx_hbm.at[wid], bidx_v)
    cdescs = [pltpu.async_copy(ones_v, cacc.at[bidx_v.at[j]], ssemA, add=True)
              for j in range(BCH)]
    for d in cdescs:
        d.wait()

    plsc.subcore_barrier()
    pltpu.sync_copy(dacc.at[pl.ds(s * RPT, RPT)],
                    degp_hbm.at[c, pl.ds(s * RPT, RPT)])
    pltpu.sync_copy(cacc.at[pl.ds(s * CRPT, CRPT)],
                    cntp_hbm.at[c, pl.ds(s * CRPT, CRPT)])


@functools.partial(
    pl.kernel,
    out_type=jax.ShapeDtypeStruct((NC, NPAD, LANES), jnp.float32),
    mesh=_MESH,
    scratch_types=[
        pltpu.VMEM_SHARED((NPAD, LANES), jnp.float32),
        pltpu.VMEM((2, K, C), jnp.int32),
        pltpu.VMEM((2, K, C), jnp.int32),
        pltpu.VMEM((2, K, C, LANES), jnp.float32),
        pltpu.SemaphoreType.DMA,
        pltpu.SemaphoreType.DMA,
        pltpu.SemaphoreType.DMA,
        pltpu.SemaphoreType.DMA,
    ],
    compiler_params=pltpu.CompilerParams(use_tc_tiling_on_sc=False),
)
def _agg_kernel(tbl_hbm, src_hbm, dst_hbm, zeros_hbm,
                out_hbm,
                acc, src_v, dst_v, rows_v, gsemA, gsemB, ssemA, ssemB):
    c = lax.axis_index("c")
    s = lax.axis_index("s")
    wid = c * NS + s

    def stage(g, slot):
        pltpu.sync_copy(src_hbm.at[wid, g], src_v.at[slot])
        pltpu.sync_copy(dst_hbm.at[wid, g], dst_v.at[slot])

    def fire_gathers(slot, gsem):
        return [pltpu.async_copy(tbl_hbm.at[src_v.at[slot, j]],
                                 rows_v.at[slot, j], gsem)
                for j in range(K)]

    def fire_scatters(slot, ssem):
        return [pltpu.async_copy(rows_v.at[slot, j],
                                 acc.at[dst_v.at[slot, j]], ssem, add=True)
                for j in range(K)]

    pltpu.sync_copy(zeros_hbm, acc.at[pl.ds(s * RPT, RPT)])
    plsc.subcore_barrier()
    stage(0, 0)
    fire_gathers(0, gsemA)

    def body(i, carry):
        stage(2 * i + 1, 1)
        gdB = fire_gathers(1, gsemB)
        gdA_wait = [pltpu.make_async_copy(tbl_hbm.at[src_v.at[0, j]],
                                          rows_v.at[0, j], gsemA)
                    for j in range(K)]
        for d in gdA_wait:
            d.wait()
        sdA = fire_scatters(0, ssemA)
        for d in sdA:
            d.wait()

        @pl.when(i < G // 2 - 1)
        def _():
            stage(2 * i + 2, 0)
            fire_gathers(0, gsemA)

        for d in gdB:
            d.wait()
        sdB = fire_scatters(1, ssemB)
        for d in sdB:
            d.wait()
        return carry

    lax.fori_loop(0, G // 2, body, 0)
    plsc.subcore_barrier()
    pltpu.sync_copy(acc.at[pl.ds(s * RPT, RPT)],
                    out_hbm.at[c, pl.ds(s * RPT, RPT)])


def _enc_body(x_ref, degp_ref, w1_ref, b1_ref, w2_ref, b2_ref, w3_ref, b3_ref,
              ha_ref, hb_ref, dinv_ref):
    degp = degp_ref[...]
    deg = degp[0, :, 0] + degp[1, :, 0]
    dinv = lax.rsqrt(jnp.maximum(deg, 1.0))
    h = jnp.maximum(jnp.dot(x_ref[...], w1_ref[...],
                            preferred_element_type=jnp.float32) + b1_ref[...], 0.0)
    h = jnp.maximum(jnp.dot(h, w2_ref[...],
                            preferred_element_type=jnp.float32) + b2_ref[...], 0.0)
    h = jnp.dot(h, w3_ref[...], preferred_element_type=jnp.float32) + b3_ref[...]
    hp = h * dinv[:, None]
    ha_ref[...] = hp[:, :LANES]
    hb_ref[...] = hp[:, LANES:]
    dinv_ref[...] = jnp.broadcast_to(dinv[:, None], (BN, LANES))


def _l1_body(pa_ref, pb_ref, dinv_ref, w_ref, b_ref, o0, o1, o2, o3):
    pa = pa_ref[...]
    pb = pb_ref[...]
    agg = jnp.concatenate([pa[0] + pa[1], pb[0] + pb[1]], axis=1)
    dinv = dinv_ref[...][:, 0]
    h = jnp.maximum(jnp.dot(agg * dinv[:, None], w_ref[...],
                            preferred_element_type=jnp.float32) + b_ref[...], 0.0)
    hp = h * dinv[:, None]
    o0[...] = hp[:, 0:16]
    o1[...] = hp[:, 16:32]
    o2[...] = hp[:, 32:48]
    o3[...] = hp[:, 48:64]


def _l2_body(p0_ref, p1_ref, p2_ref, p3_ref, dinv_ref, bidx_ref, w_ref, b_ref,
             cntp_ref, hw1_ref, hb1_ref, hw2_ref, hb2_ref, hw3_ref, hb3_ref,
             pool_ref, cost_ref):
    i = pl.program_id(0)
    parts = [p0_ref[...], p1_ref[...], p2_ref[...], p3_ref[...]]
    agg = jnp.concatenate([p[0] + p[1] for p in parts], axis=1)
    dinv = dinv_ref[...][:, 0]
    h2 = jnp.maximum(jnp.dot(agg * dinv[:, None], w_ref[...],
                             preferred_element_type=jnp.float32) + b_ref[...], 0.0)
    plans = lax.broadcasted_iota(jnp.int32, (N_PLANS, BN), 0)
    onehot = (plans == bidx_ref[0, 0][None, :]).astype(jnp.float32)
    contrib = jnp.dot(onehot, h2, preferred_element_type=jnp.float32)

    @pl.when(i == 0)
    def _():
        pool_ref[...] = contrib

    @pl.when(i > 0)
    def _():
        pool_ref[...] += contrib

    @pl.when(i == GRID_N - 1)
    def _():
        cntp = cntp_ref[...]
        counts = cntp[0, :, 0] + cntp[1, :, 0]
        emb = pool_ref[...] / jnp.maximum(counts, 1.0)[:, None]
        z = jnp.maximum(jnp.dot(emb, hw1_ref[...],
                                preferred_element_type=jnp.float32) + hb1_ref[...], 0.0)
        z = jnp.maximum(jnp.dot(z, hw2_ref[...],
                                preferred_element_type=jnp.float32) + hb2_ref[...], 0.0)
        cst = jnp.dot(z, hw3_ref[...],
                      preferred_element_type=jnp.float32) + hb3_ref[...]
        cost_ref[...] = jnp.maximum(cst, 0.0) + jnp.log1p(jnp.exp(-jnp.abs(cst)))


def _full(shape):
    return pl.BlockSpec(shape, lambda i: tuple(0 for _ in shape))


def kernel(node_features, edge_index, edge_types, batch_idx,
           ne_W1, ne_b1, ne_W2, ne_b2, ne_W3, ne_b3,
           g_W1, g_b1, g_W2, g_b2,
           h_W1, h_b1, h_W2, h_b2, h_W3, h_b3):
    del edge_types
    src = edge_index[0].astype(jnp.int32)
    dst = edge_index[1].astype(jnp.int32)
    epad = PAD_NODE + (jnp.arange(EP - N_EDGES, dtype=jnp.int32) % (NPAD - N_NODES))
    src_flat = jnp.concatenate([src, epad])
    dst_flat = jnp.concatenate([dst, epad])
    src_r = src_flat.reshape(NW, G, K, C)
    dst_r = dst_flat.reshape(NW, G, K, C)
    bpad = PAD_PLAN + (jnp.arange(NW * BPTP - N_NODES, dtype=jnp.int32)
                       % (CPAD - N_PLANS))
    bidx_r = jnp.concatenate([batch_idx.astype(jnp.int32), bpad]).reshape(NW, BCH, C)
    batch2d = batch_idx.astype(jnp.int32).reshape(GRID_N, 1, BN)
    zeros_blk = jnp.zeros((RPT, LANES), jnp.float32)
    ones_blk = jnp.ones((C, LANES), jnp.float32)

    degp, cntp = _deg_kernel(dst_r, bidx_r, zeros_blk, ones_blk)

    h0a, h0b, dinv2d = pl.pallas_call(
        _enc_body,
        grid=(GRID_N,),
        in_specs=[
            pl.BlockSpec((BN, 50), lambda i: (i, 0)),
            pl.BlockSpec((NC, BN, LANES), lambda i: (0, i, 0)),
            _full((50, 128)), _full((128,)),
            _full((128, 64)), _full((64,)),
            _full((64, 32)), _full((32,)),
        ],
        out_specs=[
            pl.BlockSpec((BN, LANES), lambda i: (i, 0)),
            pl.BlockSpec((BN, LANES), lambda i: (i, 0)),
            pl.BlockSpec((BN, LANES), lambda i: (i, 0)),
        ],
        out_shape=[
            jax.ShapeDtypeStruct((NPAD, LANES), jnp.float32),
            jax.ShapeDtypeStruct((NPAD, LANES), jnp.float32),
            jax.ShapeDtypeStruct((NPAD, LANES), jnp.float32),
        ],
    )(node_features, degp, ne_W1, ne_b1, ne_W2, ne_b2, ne_W3, ne_b3)

    p1a = _agg_kernel(h0a, src_r, dst_r, zeros_blk)
    p1b = _agg_kernel(h0b, src_r, dst_r, zeros_blk)

    h1c = pl.pallas_call(
        _l1_body,
        grid=(GRID_N,),
        in_specs=[
            pl.BlockSpec((NC, BN, LANES), lambda i: (0, i, 0)),
            pl.BlockSpec((NC, BN, LANES), lambda i: (0, i, 0)),
            pl.BlockSpec((BN, LANES), lambda i: (i, 0)),
            _full((32, 64)), _full((64,)),
        ],
        out_specs=[pl.BlockSpec((BN, LANES), lambda i: (i, 0))] * 4,
        out_shape=[jax.ShapeDtypeStruct((NPAD, LANES), jnp.float32)] * 4,
    )(p1a, p1b, dinv2d, g_W1, g_b1)

    p2 = [_agg_kernel(t, src_r, dst_r, zeros_blk) for t in h1c]

    _, cost = pl.pallas_call(
        _l2_body,
        grid=(GRID_N,),
        in_specs=[
            pl.BlockSpec((NC, BN, LANES), lambda i: (0, i, 0)),
            pl.BlockSpec((NC, BN, LANES), lambda i: (0, i, 0)),
            pl.BlockSpec((NC, BN, LANES), lambda i: (0, i, 0)),
            pl.BlockSpec((NC, BN, LANES), lambda i: (0, i, 0)),
            pl.BlockSpec((BN, LANES), lambda i: (i, 0)),
            pl.BlockSpec((1, 1, BN), lambda i: (i, 0, 0)),
            _full((64, 64)), _full((64,)),
            pl.BlockSpec((NC, N_PLANS, LANES), lambda i: (0, 0, 0)),
            _full((64, 32)), _full((32,)),
            _full((32, 16)), _full((16,)),
            _full((16, 1)), _full((1,)),
        ],
        out_specs=[
            pl.BlockSpec((N_PLANS, 64), lambda i: (0, 0)),
            pl.BlockSpec((N_PLANS, 1), lambda i: (0, 0)),
        ],
        out_shape=[
            jax.ShapeDtypeStruct((N_PLANS, 64), jnp.float32),
            jax.ShapeDtypeStruct((N_PLANS, 1), jnp.float32),
        ],
    )(p2[0], p2[1], p2[2], p2[3], dinv2d, batch2d, g_W2, g_b2, cntp,
      h_W1, h_b1, h_W2, h_b2, h_W3, h_b3)
    return cost


# speedup vs baseline: 2.1086x; 1.4037x over previous
import functools

import jax
import jax.numpy as jnp
from jax import lax
from jax.experimental import pallas as pl
from jax.experimental.pallas import tpu as pltpu
from jax.experimental.pallas import tpu_sc as plsc

N_NODES = 100000
N_EDGES = 1600000
N_PLANS = 1024

NC, NS, LANES = 2, 16, 16
W = 32
NW = NC * NS
C = 128
K = 6
G = 66
EPT = G * K * C
EP = NW * EPT
PAD_NODE = N_NODES

NPAD = 100096
RPT = NPAD // NS

BCH = 25
BPTP = BCH * C
PAD_PLAN = N_PLANS
CPAD = 1152
CRPT = CPAD // NS

BN = 2000
GRID_N = N_NODES // BN

_MESH = plsc.VectorSubcoreMesh(
    core_axis_name="c", subcore_axis_name="s", num_cores=NC, num_subcores=NS)


@functools.partial(
    pl.kernel,
    out_type=(jax.ShapeDtypeStruct((NC, NPAD, LANES), jnp.float32),
              jax.ShapeDtypeStruct((NC, CPAD, LANES), jnp.float32)),
    mesh=_MESH,
    scratch_types=[
        pltpu.VMEM_SHARED((NPAD, LANES), jnp.float32),
        pltpu.VMEM_SHARED((CPAD, LANES), jnp.float32),
        pltpu.VMEM((2, K, C), jnp.int32),
        pltpu.VMEM((BCH, C), jnp.int32),
        pltpu.VMEM((C, LANES), jnp.float32),
        pltpu.SemaphoreType.DMA,
        pltpu.SemaphoreType.DMA,
    ],
    compiler_params=pltpu.CompilerParams(use_tc_tiling_on_sc=False),
)
def _deg_kernel(dst_hbm, bidx_hbm, zeros_hbm, ones_hbm,
                degp_hbm, cntp_hbm,
                dacc, cacc, dst_v, bidx_v, ones_v, ssemA, ssemB):
    c = lax.axis_index("c")
    s = lax.axis_index("s")
    wid = c * NS + s
    pltpu.sync_copy(ones_hbm, ones_v)
    pltpu.sync_copy(zeros_hbm, dacc.at[pl.ds(s * RPT, RPT)])
    pltpu.sync_copy(zeros_hbm.at[pl.ds(0, CRPT)], cacc.at[pl.ds(s * CRPT, CRPT)])
    plsc.subcore_barrier()

    def stage(g, slot):
        pltpu.sync_copy(dst_hbm.at[wid, g], dst_v.at[slot])

    def fire(slot, sem):
        return [pltpu.async_copy(ones_v, dacc.at[dst_v.at[slot, j]], sem,
                                 add=True)
                for j in range(K)]

    def drain(slot, sem):
        for j in range(K):
            pltpu.make_async_copy(ones_v, dacc.at[dst_v.at[slot, j]],
                                  sem).wait()

    stage(0, 0)
    fire(0, ssemA)
    stage(1, 1)
    fire(1, ssemB)

    def g_body(i, carry):
        drain(0, ssemA)
        stage(2 * i, 0)
        fire(0, ssemA)
        drain(1, ssemB)
        stage(2 * i + 1, 1)
        fire(1, ssemB)
        return carry

    lax.fori_loop(1, G // 2, g_body, 0)
    drain(0, ssemA)
    drain(1, ssemB)

    pltpu.sync_copy(bidx_hbm.at[wid], bidx_v)
    cdescs = [pltpu.async_copy(ones_v, cacc.at[bidx_v.at[j]], ssemA, add=True)
              for j in range(BCH)]
    for d in cdescs:
        d.wait()

    plsc.subcore_barrier()
    pltpu.sync_copy(dacc.at[pl.ds(s * RPT, RPT)],
                    degp_hbm.at[c, pl.ds(s * RPT, RPT)])
    pltpu.sync_copy(cacc.at[pl.ds(s * CRPT, CRPT)],
                    cntp_hbm.at[c, pl.ds(s * CRPT, CRPT)])


@functools.partial(
    pl.kernel,
    out_type=jax.ShapeDtypeStruct((NC, NPAD, W), jnp.bfloat16),
    mesh=_MESH,
    scratch_types=[
        pltpu.VMEM_SHARED((NPAD, W), jnp.bfloat16),
        pltpu.VMEM((2, K, C), jnp.int32),
        pltpu.VMEM((2, K, C), jnp.int32),
        pltpu.VMEM((2, K, C, W), jnp.bfloat16),
        pltpu.SemaphoreType.DMA,
        pltpu.SemaphoreType.DMA,
        pltpu.SemaphoreType.DMA,
        pltpu.SemaphoreType.DMA,
    ],
    compiler_params=pltpu.CompilerParams(use_tc_tiling_on_sc=False),
)
def _agg_kernel(tbl_hbm, src_hbm, dst_hbm, zeros_hbm,
                out_hbm,
                acc, src_v, dst_v, rows_v, gsemA, gsemB, ssemA, ssemB):
    c = lax.axis_index("c")
    s = lax.axis_index("s")
    wid = c * NS + s

    def stage(g, slot):
        pltpu.sync_copy(src_hbm.at[wid, g], src_v.at[slot])
        pltpu.sync_copy(dst_hbm.at[wid, g], dst_v.at[slot])

    def fire_gathers(slot, gsem):
        return [pltpu.async_copy(tbl_hbm.at[src_v.at[slot, j]],
                                 rows_v.at[slot, j], gsem)
                for j in range(K)]

    def fire_scatters(slot, ssem):
        return [pltpu.async_copy(rows_v.at[slot, j],
                                 acc.at[dst_v.at[slot, j]], ssem, add=True)
                for j in range(K)]

    pltpu.sync_copy(zeros_hbm, acc.at[pl.ds(s * RPT, RPT)])
    plsc.subcore_barrier()
    stage(0, 0)
    fire_gathers(0, gsemA)

    def body(i, carry):
        stage(2 * i + 1, 1)
        gdB = fire_gathers(1, gsemB)
        gdA_wait = [pltpu.make_async_copy(tbl_hbm.at[src_v.at[0, j]],
                                          rows_v.at[0, j], gsemA)
                    for j in range(K)]
        for d in gdA_wait:
            d.wait()
        sdA = fire_scatters(0, ssemA)
        for d in sdA:
            d.wait()

        @pl.when(i < G // 2 - 1)
        def _():
            stage(2 * i + 2, 0)
            fire_gathers(0, gsemA)

        for d in gdB:
            d.wait()
        sdB = fire_scatters(1, ssemB)
        for d in sdB:
            d.wait()
        return carry

    lax.fori_loop(0, G // 2, body, 0)
    plsc.subcore_barrier()
    pltpu.sync_copy(acc.at[pl.ds(s * RPT, RPT)],
                    out_hbm.at[c, pl.ds(s * RPT, RPT)])


def _enc_body(x_ref, degp_ref, w1_ref, b1_ref, w2_ref, b2_ref, w3_ref, b3_ref,
              ht_ref, dinv_ref):
    degp = degp_ref[...]
    deg = degp[0, :, 0] + degp[1, :, 0]
    dinv = lax.rsqrt(jnp.maximum(deg, 1.0))
    h = jnp.maximum(jnp.dot(x_ref[...], w1_ref[...],
                            preferred_element_type=jnp.float32) + b1_ref[...], 0.0)
    h = jnp.maximum(jnp.dot(h, w2_ref[...],
                            preferred_element_type=jnp.float32) + b2_ref[...], 0.0)
    h = jnp.dot(h, w3_ref[...], preferred_element_type=jnp.float32) + b3_ref[...]
    hp = h * dinv[:, None]
    ht_ref[...] = hp.astype(jnp.bfloat16)
    dinv_ref[...] = jnp.broadcast_to(dinv[:, None], (BN, LANES))


def _l1_body(p_ref, dinv_ref, w_ref, b_ref, o0, o1):
    p = p_ref[...].astype(jnp.float32)
    agg = p[0] + p[1]
    dinv = dinv_ref[...][:, 0]
    h = jnp.maximum(jnp.dot(agg * dinv[:, None], w_ref[...],
                            preferred_element_type=jnp.float32) + b_ref[...], 0.0)
    hp = h * dinv[:, None]
    o0[...] = hp[:, 0:W].astype(jnp.bfloat16)
    o1[...] = hp[:, W:].astype(jnp.bfloat16)


def _l2_body(p0_ref, p1_ref, dinv_ref, bidx_ref, w_ref, b_ref,
             cntp_ref, hw1_ref, hb1_ref, hw2_ref, hb2_ref, hw3_ref, hb3_ref,
             pool_ref, cost_ref):
    i = pl.program_id(0)
    parts = [p0_ref[...].astype(jnp.float32), p1_ref[...].astype(jnp.float32)]
    agg = jnp.concatenate([p[0] + p[1] for p in parts], axis=1)
    dinv = dinv_ref[...][:, 0]
    h2 = jnp.maximum(jnp.dot(agg * dinv[:, None], w_ref[...],
                             preferred_element_type=jnp.float32) + b_ref[...], 0.0)
    plans = lax.broadcasted_iota(jnp.int32, (N_PLANS, BN), 0)
    onehot = (plans == bidx_ref[0, 0][None, :]).astype(jnp.float32)
    contrib = jnp.dot(onehot, h2, preferred_element_type=jnp.float32)

    @pl.when(i == 0)
    def _():
        pool_ref[...] = contrib

    @pl.when(i > 0)
    def _():
        pool_ref[...] += contrib

    @pl.when(i == GRID_N - 1)
    def _():
        cntp = cntp_ref[...]
        counts = cntp[0, :, 0] + cntp[1, :, 0]
        emb = pool_ref[...] / jnp.maximum(counts, 1.0)[:, None]
        z = jnp.maximum(jnp.dot(emb, hw1_ref[...],
                                preferred_element_type=jnp.float32) + hb1_ref[...], 0.0)
        z = jnp.maximum(jnp.dot(z, hw2_ref[...],
                                preferred_element_type=jnp.float32) + hb2_ref[...], 0.0)
        cst = jnp.dot(z, hw3_ref[...],
                      preferred_element_type=jnp.float32) + hb3_ref[...]
        cost_ref[...] = jnp.maximum(cst, 0.0) + jnp.log1p(jnp.exp(-jnp.abs(cst)))


def _full(shape):
    return pl.BlockSpec(shape, lambda i: tuple(0 for _ in shape))


def kernel(node_features, edge_index, edge_types, batch_idx,
           ne_W1, ne_b1, ne_W2, ne_b2, ne_W3, ne_b3,
           g_W1, g_b1, g_W2, g_b2,
           h_W1, h_b1, h_W2, h_b2, h_W3, h_b3):
    del edge_types
    src = edge_index[0].astype(jnp.int32)
    dst = edge_index[1].astype(jnp.int32)
    epad = PAD_NODE + (jnp.arange(EP - N_EDGES, dtype=jnp.int32) % (NPAD - N_NODES))
    src_flat = jnp.concatenate([src, epad])
    dst_flat = jnp.concatenate([dst, epad])
    src_r = src_flat.reshape(NW, G, K, C)
    dst_r = dst_flat.reshape(NW, G, K, C)
    bpad = PAD_PLAN + (jnp.arange(NW * BPTP - N_NODES, dtype=jnp.int32)
                       % (CPAD - N_PLANS))
    bidx_r = jnp.concatenate([batch_idx.astype(jnp.int32), bpad]).reshape(NW, BCH, C)
    batch2d = batch_idx.astype(jnp.int32).reshape(GRID_N, 1, BN)
    zeros_blk = jnp.zeros((RPT, LANES), jnp.float32)
    zeros_blk16 = jnp.zeros((RPT, W), jnp.bfloat16)
    ones_blk = jnp.ones((C, LANES), jnp.float32)

    degp, cntp = _deg_kernel(dst_r, bidx_r, zeros_blk, ones_blk)

    h0t, dinv2d = pl.pallas_call(
        _enc_body,
        grid=(GRID_N,),
        in_specs=[
            pl.BlockSpec((BN, 50), lambda i: (i, 0)),
            pl.BlockSpec((NC, BN, LANES), lambda i: (0, i, 0)),
            _full((50, 128)), _full((128,)),
            _full((128, 64)), _full((64,)),
            _full((64, 32)), _full((32,)),
        ],
        out_specs=[
            pl.BlockSpec((BN, W), lambda i: (i, 0)),
            pl.BlockSpec((BN, LANES), lambda i: (i, 0)),
        ],
        out_shape=[
            jax.ShapeDtypeStruct((NPAD, W), jnp.bfloat16),
            jax.ShapeDtypeStruct((NPAD, LANES), jnp.float32),
        ],
    )(node_features, degp, ne_W1, ne_b1, ne_W2, ne_b2, ne_W3, ne_b3)

    p1 = _agg_kernel(h0t, src_r, dst_r, zeros_blk16)

    h1c = pl.pallas_call(
        _l1_body,
        grid=(GRID_N,),
        in_specs=[
            pl.BlockSpec((NC, BN, W), lambda i: (0, i, 0)),
            pl.BlockSpec((BN, LANES), lambda i: (i, 0)),
            _full((32, 64)), _full((64,)),
        ],
        out_specs=[pl.BlockSpec((BN, W), lambda i: (i, 0))] * 2,
        out_shape=[jax.ShapeDtypeStruct((NPAD, W), jnp.bfloat16)] * 2,
    )(p1, dinv2d, g_W1, g_b1)

    p2 = [_agg_kernel(t, src_r, dst_r, zeros_blk16) for t in h1c]

    _, cost = pl.pallas_call(
        _l2_body,
        grid=(GRID_N,),
        in_specs=[
            pl.BlockSpec((NC, BN, W), lambda i: (0, i, 0)),
            pl.BlockSpec((NC, BN, W), lambda i: (0, i, 0)),
            pl.BlockSpec((BN, LANES), lambda i: (i, 0)),
            pl.BlockSpec((1, 1, BN), lambda i: (i, 0, 0)),
            _full((64, 64)), _full((64,)),
            pl.BlockSpec((NC, N_PLANS, LANES), lambda i: (0, 0, 0)),
            _full((64, 32)), _full((32,)),
            _full((32, 16)), _full((16,)),
            _full((16, 1)), _full((1,)),
        ],
        out_specs=[
            pl.BlockSpec((N_PLANS, 64), lambda i: (0, 0)),
            pl.BlockSpec((N_PLANS, 1), lambda i: (0, 0)),
        ],
        out_shape=[
            jax.ShapeDtypeStruct((N_PLANS, 64), jnp.float32),
            jax.ShapeDtypeStruct((N_PLANS, 1), jnp.float32),
        ],
    )(p2[0], p2[1], dinv2d, batch2d, g_W2, g_b2, cntp,
      h_W1, h_b1, h_W2, h_b2, h_W3, h_b3)
    return cost
